# baseline (speedup 1.0000x reference)
import functools

import jax
import jax.numpy as jnp
from jax import lax
from jax.experimental import pallas as pl
from jax.experimental.pallas import tpu as pltpu
from jax.experimental.pallas import tpu_sc as plsc

V = 32
B = 2
N = 16384
NT = 16
PPT = N // NT
NB = V * V * V
BPT = NB // NT
L = 16


_NBR_ELEMS = V * V * V * 27 * 3
_NBR_CHUNK = _NBR_ELEMS // NT


def _sc_sort_body(pc_ref, nvl_ref, codes_out, idx_out, mask_out, xchg_hbm,
                  nbr_out,
                  xv, yv, zv, key_v, cnt_v, col2_v, pfx_v,
                  fmask_v, tot_v, h2_v, pos_v, cval_v, ival_v, dsem, nsem,
                  cnt_sp, outc_sp, outi_sp):
  c = lax.axis_index("c")
  t = lax.axis_index("s")
  iota = lax.iota(jnp.int32, L)
  zeros = jnp.zeros((L,), jnp.int32)

  nsl = pl.ds(t * _NBR_CHUNK, _NBR_CHUNK)
  nbr_desc = pltpu.async_copy(
      nvl_ref.at[nsl], nbr_out.at[pl.ds(c * _NBR_ELEMS + t * _NBR_CHUNK,
                                        _NBR_CHUNK)], nsem)

  def zero_key(i, _):
    key_v[pl.ds(i * L, L)] = zeros
    return 0
  lax.fori_loop(0, PPT // L, zero_key, 0)
  pltpu.sync_copy(key_v, outc_sp.at[pl.ds(t * PPT, PPT)])
  pltpu.sync_copy(key_v, outi_sp.at[pl.ds(t * PPT, PPT)])

  pltpu.sync_copy(pc_ref.at[pl.ds((c * 3 + 0) * N + t * PPT, PPT)], xv)
  pltpu.sync_copy(pc_ref.at[pl.ds((c * 3 + 1) * N + t * PPT, PPT)], yv)
  pltpu.sync_copy(pc_ref.at[pl.ds((c * 3 + 2) * N + t * PPT, PPT)], zv)

  scale = jnp.float32(V - 1)

  def compute_keys(i, _):
    sl = pl.ds(i * L, L)
    ix = (xv[sl] * scale).astype(jnp.int32)
    iy = (yv[sl] * scale).astype(jnp.int32)
    iz = (zv[sl] * scale).astype(jnp.int32)
    key_v[sl] = (ix * V + iy) * V + iz
    return 0
  lax.fori_loop(0, PPT // L, compute_keys, 0)

  def zero_cnt(i, _):
    sl = pl.ds(i * 8 * L, L)
    base = i * 8 * L
    for u in range(8):
      cnt_v[pl.ds(base + u * L, L)] = zeros
    return 0
  lax.fori_loop(0, NB // (8 * L), zero_cnt, 0)

  def hist(i, _):
    k16 = key_v[pl.ds(i * L, L)]
    cnt16, last16 = plsc.scan_count(k16)
    plsc.addupdate_scatter(cnt_v, [k16], cnt16, mask=last16)
    return 0
  lax.fori_loop(0, PPT // L, hist, 0)

  pltpu.sync_copy(cnt_v, cnt_sp.at[t])
  plsc.subcore_barrier()

  descs = [pltpu.async_copy(cnt_sp.at[tp, pl.ds(t * BPT, BPT)],
                            col2_v.at[tp], dsem) for tp in range(NT)]
  for d in descs:
    d.wait()

  def pass1(j, carry):
    sl = pl.ds(j * L, L)
    tot16 = col2_v[0, sl]
    for tp in range(1, NT):
      tot16 = tot16 + col2_v[tp, sl]
    fmask_v[sl] = jnp.where(tot16 > 0, jnp.float32(1.0), jnp.float32(0.0))
    s = plsc.cumsum(tot16)
    pfx_v[sl] = s - tot16 + jnp.full((L,), carry, jnp.int32)
    return carry + jnp.max(s)
  t_total = lax.fori_loop(0, BPT // L, pass1, jnp.int32(0))
  pltpu.sync_copy(fmask_v, mask_out.at[pl.ds(c * NB + t * BPT, BPT)])

  tot_v[...] = jnp.full((L,), t_total, jnp.int32)
  pltpu.sync_copy(tot_v, xchg_hbm.at[pl.ds((c * NT + t) * L, L)])
  plsc.subcore_barrier()
  pltpu.sync_copy(xchg_hbm.at[pl.ds(c * NT * L, NT * L)], h2_v)
  diag = plsc.load_gather(h2_v, [iota * (L + 1)])
  gbase = jnp.sum(jnp.where(iota < t, diag, 0))

  def pass2(j, _):
    sl = pl.ds(j * L, L)
    acc16 = pfx_v[sl] + jnp.full((L,), gbase, jnp.int32)
    for tp in range(NT):
      nxt = acc16 + col2_v[tp, sl]
      col2_v[tp, sl] = acc16
      acc16 = nxt
    return 0
  lax.fori_loop(0, BPT // L, pass2, 0)
  descs = [pltpu.async_copy(col2_v.at[tp],
                            cnt_sp.at[tp, pl.ds(t * BPT, BPT)], dsem)
           for tp in range(NT)]
  for d in descs:
    d.wait()
  plsc.subcore_barrier()

  pltpu.sync_copy(cnt_sp.at[t], cnt_v)

  for g in range(8):
    def rank_chunk(i8, _, g=g):
      i = g * 8 + i8
      sl = pl.ds(i * L, L)
      gsl = pl.ds(i8 * L, L)
      k16 = key_v[sl]
      cnt16, last16 = plsc.scan_count(k16)
      base16 = plsc.load_gather(cnt_v, [k16])
      plsc.addupdate_scatter(cnt_v, [k16], cnt16, mask=last16)
      pos_v[g, gsl] = base16 + cnt16 - 1
      ix = k16 >> 10
      iy = (k16 >> 5) & 31
      iz = k16 & 31
      cval_v[g, gsl] = ix * 10000 + iy * 100 + iz
      ival_v[g, gsl] = t * PPT + i * L + iota
      return 0
    lax.fori_loop(0, 8, rank_chunk, 0)
    pltpu.sync_copy(cval_v.at[g], outc_sp.at[pos_v.at[g]], add=True)
    pltpu.sync_copy(ival_v.at[g], outi_sp.at[pos_v.at[g]], add=True)
  plsc.subcore_barrier()

  sl = pl.ds(t * PPT, PPT)
  osl = pl.ds(c * N + t * PPT, PPT)
  pltpu.sync_copy(outc_sp.at[sl], codes_out.at[osl])
  pltpu.sync_copy(outi_sp.at[sl], idx_out.at[osl])
  nbr_desc.wait()


_sc_sort = pl.kernel(
    _sc_sort_body,
    out_type=[
        jax.ShapeDtypeStruct((B * N,), jnp.int32),
        jax.ShapeDtypeStruct((B * N,), jnp.int32),
        jax.ShapeDtypeStruct((B * NB,), jnp.float32),
        jax.ShapeDtypeStruct((B * NT * L,), jnp.int32),
        jax.ShapeDtypeStruct((B * _NBR_ELEMS,), jnp.float32),
    ],
    mesh=plsc.VectorSubcoreMesh(core_axis_name="c", subcore_axis_name="s"),
    compiler_params=pltpu.CompilerParams(needs_layout_passes=False),
    scratch_types=[
        pltpu.VMEM((PPT,), jnp.float32),
        pltpu.VMEM((PPT,), jnp.float32),
        pltpu.VMEM((PPT,), jnp.float32),
        pltpu.VMEM((PPT,), jnp.int32),
        pltpu.VMEM((NB,), jnp.int32),
        pltpu.VMEM((NT, BPT), jnp.int32),
        pltpu.VMEM((BPT,), jnp.int32),
        pltpu.VMEM((BPT,), jnp.float32),
        pltpu.VMEM((L,), jnp.int32),
        pltpu.VMEM((NT * L,), jnp.int32),
        pltpu.VMEM((8, 128), jnp.int32),
        pltpu.VMEM((8, 128), jnp.int32),
        pltpu.VMEM((8, 128), jnp.int32),
        pltpu.SemaphoreType.DMA,
        pltpu.SemaphoreType.DMA,
        pltpu.VMEM_SHARED((NT, NB), jnp.int32),
        pltpu.VMEM_SHARED((N,), jnp.int32),
        pltpu.VMEM_SHARED((N,), jnp.int32),
    ],
)


def _nbr_copy_body(in_ref, out_ref):
  out_ref[0] = in_ref[...]


_NBR_ROWS = V * V * V * 27 * 3 // 1024


def _nbr_broadcast(nbr_flat):
  rows_per_block = _NBR_ROWS // 6
  return pl.pallas_call(
      _nbr_copy_body,
      grid=(B, 6),
      in_specs=[pl.BlockSpec((rows_per_block, 1024), lambda b, i: (i, 0))],
      out_specs=pl.BlockSpec((1, rows_per_block, 1024), lambda b, i: (b, i, 0)),
      out_shape=jax.ShapeDtypeStruct((B, _NBR_ROWS, 1024), jnp.float32),
  )(nbr_flat)


@jax.jit
def kernel(point_cloud, neighbour_voxel_list):
  pc_t = jnp.transpose(point_cloud, (0, 2, 1)).reshape(-1)
  sorted_codes, sorted_idx, mask_flat, _, nbr_flat = _sc_sort(
      pc_t, neighbour_voxel_list.reshape(-1))
  nbr = nbr_flat.reshape((B,) + neighbour_voxel_list.shape)
  mask = mask_flat.reshape(B, V, V, V)
  return sorted_codes.reshape(B, N), sorted_idx.reshape(B, N), nbr, mask


# speedup vs baseline: 51.6092x; 51.6092x over previous
import functools

import jax
import jax.numpy as jnp
from jax import lax
from jax.experimental import pallas as pl
from jax.experimental.pallas import tpu as pltpu
from jax.experimental.pallas import tpu_sc as plsc

V = 32
B = 2
N = 16384
NT = 16
PPT = N // NT
NB = V * V * V
BPT = NB // NT
L = 16


def _sc_sort_body(pc_ref, codes_out, idx_out, mask_out, xchg_hbm,
                  xv, yv, zv, key_v, cnt_v, col2_v, pfx_v,
                  fmask_v, tot_v, h2_v, pos_v, cval_v, ival_v, dsem,
                  cnt_sp, outc_sp, outi_sp):
  c = lax.axis_index("c")
  t = lax.axis_index("s")
  iota = lax.iota(jnp.int32, L)
  zeros = jnp.zeros((L,), jnp.int32)

  def zero_key(i, _):
    key_v[pl.ds(i * L, L)] = zeros
    return 0
  lax.fori_loop(0, PPT // L, zero_key, 0)
  pltpu.sync_copy(key_v, outc_sp.at[pl.ds(t * PPT, PPT)])
  pltpu.sync_copy(key_v, outi_sp.at[pl.ds(t * PPT, PPT)])

  pltpu.sync_copy(pc_ref.at[pl.ds((c * 3 + 0) * N + t * PPT, PPT)], xv)
  pltpu.sync_copy(pc_ref.at[pl.ds((c * 3 + 1) * N + t * PPT, PPT)], yv)
  pltpu.sync_copy(pc_ref.at[pl.ds((c * 3 + 2) * N + t * PPT, PPT)], zv)

  scale = jnp.float32(V - 1)

  def compute_keys(i, _):
    sl = pl.ds(i * L, L)
    ix = (xv[sl] * scale).astype(jnp.int32)
    iy = (yv[sl] * scale).astype(jnp.int32)
    iz = (zv[sl] * scale).astype(jnp.int32)
    key_v[sl] = (ix * V + iy) * V + iz
    return 0
  lax.fori_loop(0, PPT // L, compute_keys, 0, unroll=4)

  def zero_cnt(i, _):
    base = i * 16 * L
    for u in range(16):
      cnt_v[pl.ds(base + u * L, L)] = zeros
    return 0
  lax.fori_loop(0, NB // (16 * L), zero_cnt, 0)

  def hist(i, _):
    k16 = key_v[pl.ds(i * L, L)]
    cnt16, last16 = plsc.scan_count(k16)
    plsc.addupdate_scatter(cnt_v, [k16], cnt16, mask=last16)
    return 0
  lax.fori_loop(0, PPT // L, hist, 0, unroll=4)

  pltpu.sync_copy(cnt_v, cnt_sp.at[t])
  plsc.subcore_barrier()

  descs = [pltpu.async_copy(cnt_sp.at[tp, pl.ds(t * BPT, BPT)],
                            col2_v.at[tp], dsem) for tp in range(NT)]
  for d in descs:
    d.wait()

  def pass1(j, carry):
    sl = pl.ds(j * L, L)
    tot16 = col2_v[0, sl]
    for tp in range(1, NT):
      tot16 = tot16 + col2_v[tp, sl]
    fmask_v[sl] = jnp.where(tot16 > 0, jnp.float32(1.0), jnp.float32(0.0))
    s = plsc.cumsum(tot16)
    pfx_v[sl] = s - tot16 + jnp.full((L,), carry, jnp.int32)
    return carry + jnp.max(s)
  t_total = lax.fori_loop(0, BPT // L, pass1, jnp.int32(0))
  pltpu.sync_copy(fmask_v, mask_out.at[pl.ds(c * NB + t * BPT, BPT)])

  tot_v[...] = jnp.full((L,), t_total, jnp.int32)
  pltpu.sync_copy(tot_v, xchg_hbm.at[pl.ds((c * NT + t) * L, L)])
  plsc.subcore_barrier()
  pltpu.sync_copy(xchg_hbm.at[pl.ds(c * NT * L, NT * L)], h2_v)
  diag = plsc.load_gather(h2_v, [iota * (L + 1)])
  gbase = jnp.sum(jnp.where(iota < t, diag, 0))

  def pass2(j, _):
    sl = pl.ds(j * L, L)
    acc16 = pfx_v[sl] + jnp.full((L,), gbase, jnp.int32)
    for tp in range(NT):
      nxt = acc16 + col2_v[tp, sl]
      col2_v[tp, sl] = acc16
      acc16 = nxt
    return 0
  lax.fori_loop(0, BPT // L, pass2, 0)
  descs = [pltpu.async_copy(col2_v.at[tp],
                            cnt_sp.at[tp, pl.ds(t * BPT, BPT)], dsem)
           for tp in range(NT)]
  for d in descs:
    d.wait()
  plsc.subcore_barrier()

  pltpu.sync_copy(cnt_sp.at[t], cnt_v)

  for g in range(8):
    def rank_chunk(i8, _, g=g):
      i = g * 8 + i8
      sl = pl.ds(i * L, L)
      gsl = pl.ds(i8 * L, L)
      k16 = key_v[sl]
      cnt16, last16 = plsc.scan_count(k16)
      base16 = plsc.load_gather(cnt_v, [k16])
      plsc.addupdate_scatter(cnt_v, [k16], cnt16, mask=last16)
      pos_v[g, gsl] = base16 + cnt16 - 1
      ix = k16 >> 10
      iy = (k16 >> 5) & 31
      iz = k16 & 31
      cval_v[g, gsl] = ix * 10000 + iy * 100 + iz
      ival_v[g, gsl] = t * PPT + i * L + iota
      return 0
    lax.fori_loop(0, 8, rank_chunk, 0, unroll=2)
    pltpu.sync_copy(cval_v.at[g], outc_sp.at[pos_v.at[g]], add=True)
    pltpu.sync_copy(ival_v.at[g], outi_sp.at[pos_v.at[g]], add=True)
  plsc.subcore_barrier()

  sl = pl.ds(t * PPT, PPT)
  osl = pl.ds(c * N + t * PPT, PPT)
  pltpu.sync_copy(outc_sp.at[sl], codes_out.at[osl])
  pltpu.sync_copy(outi_sp.at[sl], idx_out.at[osl])


_sc_sort = pl.kernel(
    _sc_sort_body,
    out_type=[
        jax.ShapeDtypeStruct((B * N,), jnp.int32),
        jax.ShapeDtypeStruct((B * N,), jnp.int32),
        jax.ShapeDtypeStruct((B * NB,), jnp.float32),
        jax.ShapeDtypeStruct((B * NT * L,), jnp.int32),
    ],
    mesh=plsc.VectorSubcoreMesh(core_axis_name="c", subcore_axis_name="s"),
    compiler_params=pltpu.CompilerParams(needs_layout_passes=False),
    scratch_types=[
        pltpu.VMEM((PPT,), jnp.float32),
        pltpu.VMEM((PPT,), jnp.float32),
        pltpu.VMEM((PPT,), jnp.float32),
        pltpu.VMEM((PPT,), jnp.int32),
        pltpu.VMEM((NB,), jnp.int32),
        pltpu.VMEM((NT, BPT), jnp.int32),
        pltpu.VMEM((BPT,), jnp.int32),
        pltpu.VMEM((BPT,), jnp.float32),
        pltpu.VMEM((L,), jnp.int32),
        pltpu.VMEM((NT * L,), jnp.int32),
        pltpu.VMEM((8, 128), jnp.int32),
        pltpu.VMEM((8, 128), jnp.int32),
        pltpu.VMEM((8, 128), jnp.int32),
        pltpu.SemaphoreType.DMA,
        pltpu.VMEM_SHARED((NT, NB), jnp.int32),
        pltpu.VMEM_SHARED((N,), jnp.int32),
        pltpu.VMEM_SHARED((N,), jnp.int32),
    ],
)


@jax.jit
def kernel(point_cloud, neighbour_voxel_list):
  nbr = jnp.broadcast_to(
      neighbour_voxel_list[None], (B,) + neighbour_voxel_list.shape)
  pc_t = jnp.transpose(point_cloud, (0, 2, 1)).reshape(-1)
  sorted_codes, sorted_idx, mask_flat, _ = _sc_sort(pc_t)
  mask = mask_flat.reshape(B, V, V, V)
  return sorted_codes.reshape(B, N), sorted_idx.reshape(B, N), nbr, mask


# speedup vs baseline: 52.5321x; 1.0179x over previous
import functools

import jax
import jax.numpy as jnp
from jax import lax
from jax.experimental import pallas as pl
from jax.experimental.pallas import tpu as pltpu
from jax.experimental.pallas import tpu_sc as plsc

V = 32
B = 2
N = 16384
NT = 16
PPT = N // NT
NB = V * V * V
BPT = NB // NT
L = 16


def _sc_sort_body(pc_ref, codes_out, idx_out, mask_out, xchg_hbm,
                  xv, yv, zv, key_v, cnt_v, col2_v, pfx_v,
                  fmask_v, tot_v, h2_v, pos_v, cval_v, ival_v, dsem, msem,
                  cnt_sp, outc_sp, outi_sp):
  c = lax.axis_index("c")
  t = lax.axis_index("s")
  iota = lax.iota(jnp.int32, L)
  zeros = jnp.zeros((L,), jnp.int32)

  def zero_key(i, _):
    key_v[pl.ds(i * L, L)] = zeros
    return 0
  lax.fori_loop(0, PPT // L, zero_key, 0)
  pltpu.sync_copy(key_v, outc_sp.at[pl.ds(t * PPT, PPT)])
  pltpu.sync_copy(key_v, outi_sp.at[pl.ds(t * PPT, PPT)])

  pltpu.sync_copy(pc_ref.at[pl.ds((c * 3 + 0) * N + t * PPT, PPT)], xv)
  pltpu.sync_copy(pc_ref.at[pl.ds((c * 3 + 1) * N + t * PPT, PPT)], yv)
  pltpu.sync_copy(pc_ref.at[pl.ds((c * 3 + 2) * N + t * PPT, PPT)], zv)

  scale = jnp.float32(V - 1)

  def compute_keys(i, _):
    sl = pl.ds(i * L, L)
    ix = (xv[sl] * scale).astype(jnp.int32)
    iy = (yv[sl] * scale).astype(jnp.int32)
    iz = (zv[sl] * scale).astype(jnp.int32)
    key_v[sl] = (ix * V + iy) * V + iz
    return 0
  lax.fori_loop(0, PPT // L, compute_keys, 0, unroll=4)

  def zero_cnt(i, _):
    base = i * 16 * L
    for u in range(16):
      cnt_v[pl.ds(base + u * L, L)] = zeros
    return 0
  lax.fori_loop(0, NB // (16 * L), zero_cnt, 0)

  def hist(i, _):
    k16 = key_v[pl.ds(i * L, L)]
    cnt16, last16 = plsc.scan_count(k16)
    plsc.addupdate_scatter(cnt_v, [k16], cnt16, mask=last16)
    return 0
  lax.fori_loop(0, PPT // L, hist, 0, unroll=4)

  pltpu.sync_copy(cnt_v, cnt_sp.at[t])
  plsc.subcore_barrier()

  descs = [pltpu.async_copy(cnt_sp.at[tp, pl.ds(t * BPT, BPT)],
                            col2_v.at[tp], dsem) for tp in range(NT)]
  for d in descs:
    d.wait()

  def pass1(j, carry):
    sl = pl.ds(j * L, L)
    tot16 = col2_v[0, sl]
    for tp in range(1, NT):
      tot16 = tot16 + col2_v[tp, sl]
    fmask_v[sl] = jnp.where(tot16 > 0, jnp.float32(1.0), jnp.float32(0.0))
    s = plsc.cumsum(tot16)
    pfx_v[sl] = s - tot16 + jnp.full((L,), carry, jnp.int32)
    return carry + jnp.max(s)
  t_total = lax.fori_loop(0, BPT // L, pass1, jnp.int32(0))
  mask_desc = pltpu.async_copy(
      fmask_v, mask_out.at[pl.ds(c * NB + t * BPT, BPT)], msem)

  tot_v[...] = jnp.full((L,), t_total, jnp.int32)
  pltpu.sync_copy(tot_v, xchg_hbm.at[pl.ds((c * NT + t) * L, L)])
  plsc.subcore_barrier()
  pltpu.sync_copy(xchg_hbm.at[pl.ds(c * NT * L, NT * L)], h2_v)
  diag = plsc.load_gather(h2_v, [iota * (L + 1)])
  gbase = jnp.sum(jnp.where(iota < t, diag, 0))

  def pass2(j, _):
    sl = pl.ds(j * L, L)
    acc16 = pfx_v[sl] + jnp.full((L,), gbase, jnp.int32)
    for tp in range(NT):
      nxt = acc16 + col2_v[tp, sl]
      col2_v[tp, sl] = acc16
      acc16 = nxt
    return 0
  lax.fori_loop(0, BPT // L, pass2, 0)
  descs = [pltpu.async_copy(col2_v.at[tp],
                            cnt_sp.at[tp, pl.ds(t * BPT, BPT)], dsem)
           for tp in range(NT)]
  for d in descs:
    d.wait()
  plsc.subcore_barrier()

  pltpu.sync_copy(cnt_sp.at[t], cnt_v)

  sc_descs = []
  for g in range(8):
    def rank_chunk(i8, _, g=g):
      i = g * 8 + i8
      sl = pl.ds(i * L, L)
      gsl = pl.ds(i8 * L, L)
      k16 = key_v[sl]
      cnt16, last16 = plsc.scan_count(k16)
      base16 = plsc.load_gather(cnt_v, [k16])
      plsc.addupdate_scatter(cnt_v, [k16], cnt16, mask=last16)
      pos_v[g, gsl] = base16 + cnt16 - 1
      ix = k16 >> 10
      iy = (k16 >> 5) & 31
      iz = k16 & 31
      cval_v[g, gsl] = ix * 10000 + iy * 100 + iz
      ival_v[g, gsl] = t * PPT + i * L + iota
      return 0
    lax.fori_loop(0, 8, rank_chunk, 0, unroll=2)
    sc_descs.append(pltpu.async_copy(
        cval_v.at[g], outc_sp.at[pos_v.at[g]], dsem, add=True))
    sc_descs.append(pltpu.async_copy(
        ival_v.at[g], outi_sp.at[pos_v.at[g]], dsem, add=True))
  for d in sc_descs:
    d.wait()
  mask_desc.wait()
  plsc.subcore_barrier()

  sl = pl.ds(t * PPT, PPT)
  osl = pl.ds(c * N + t * PPT, PPT)
  pltpu.sync_copy(outc_sp.at[sl], codes_out.at[osl])
  pltpu.sync_copy(outi_sp.at[sl], idx_out.at[osl])


_sc_sort = pl.kernel(
    _sc_sort_body,
    out_type=[
        jax.ShapeDtypeStruct((B * N,), jnp.int32),
        jax.ShapeDtypeStruct((B * N,), jnp.int32),
        jax.ShapeDtypeStruct((B * NB,), jnp.float32),
        jax.ShapeDtypeStruct((B * NT * L,), jnp.int32),
    ],
    mesh=plsc.VectorSubcoreMesh(core_axis_name="c", subcore_axis_name="s"),
    compiler_params=pltpu.CompilerParams(needs_layout_passes=False),
    scratch_types=[
        pltpu.VMEM((PPT,), jnp.float32),
        pltpu.VMEM((PPT,), jnp.float32),
        pltpu.VMEM((PPT,), jnp.float32),
        pltpu.VMEM((PPT,), jnp.int32),
        pltpu.VMEM((NB,), jnp.int32),
        pltpu.VMEM((NT, BPT), jnp.int32),
        pltpu.VMEM((BPT,), jnp.int32),
        pltpu.VMEM((BPT,), jnp.float32),
        pltpu.VMEM((L,), jnp.int32),
        pltpu.VMEM((NT * L,), jnp.int32),
        pltpu.VMEM((8, 128), jnp.int32),
        pltpu.VMEM((8, 128), jnp.int32),
        pltpu.VMEM((8, 128), jnp.int32),
        pltpu.SemaphoreType.DMA,
        pltpu.SemaphoreType.DMA,
        pltpu.VMEM_SHARED((NT, NB), jnp.int32),
        pltpu.VMEM_SHARED((N,), jnp.int32),
        pltpu.VMEM_SHARED((N,), jnp.int32),
    ],
)


@jax.jit
def kernel(point_cloud, neighbour_voxel_list):
  nbr = jnp.broadcast_to(
      neighbour_voxel_list[None], (B,) + neighbour_voxel_list.shape)
  pc_t = jnp.transpose(point_cloud, (0, 2, 1)).reshape(-1)
  sorted_codes, sorted_idx, mask_flat, _ = _sc_sort(pc_t)
  mask = mask_flat.reshape(B, V, V, V)
  return sorted_codes.reshape(B, N), sorted_idx.reshape(B, N), nbr, mask
